# Initial kernel scaffold; baseline (speedup 1.0000x reference)
import jax
import jax.numpy as jnp
from jax.experimental import pallas as pl


def kernel(x, edge_index, We, be, Wc1, bc1, Wu1, bu1, Wc2, bc2, Wu2, bu2, Wc3, bc3, Wu3, bu3):
    raise NotImplementedError("write your pallas kernel here")


# speedup vs baseline: 6.8865x; 6.8865x over previous
import functools

import jax
import jax.numpy as jnp
from jax import lax
from jax.experimental import pallas as pl
from jax.experimental.pallas import tpu as pltpu
from jax.experimental.pallas import tpu_sc as plsc

N = 10000
E = 160000
D = 256

NC = 2
NS = 16
EPT = E // NS
K = 128
EPAD = 10240
NBAT = EPAD // K
CW = 128
ACCR = 10240
TRASH = 10200
RPT = ACCR // NS

_mesh = plsc.VectorSubcoreMesh(core_axis_name="c", subcore_axis_name="s")

f32 = jnp.float32
i32 = jnp.int32


@functools.partial(
    pl.kernel,
    out_type=(
        jax.ShapeDtypeStruct((NS, NBAT, K), i32),
        jax.ShapeDtypeStruct((NS, NBAT, K), i32),
    ),
    mesh=_mesh,
    scratch_types=[
        pltpu.VMEM((NBAT, K), i32),
        pltpu.VMEM((NBAT, K), i32),
        pltpu.VMEM((NBAT, K), i32),
        pltpu.VMEM((NBAT, K), i32),
    ],
)
def _prep(src_hbm, dst_hbm, srcl_hbm, dstl_hbm, raw_src, raw_dst, src2, dst2):
    c = lax.axis_index("c")
    s = lax.axis_index("s")

    pltpu.sync_copy(src_hbm.at[s], raw_src)
    pltpu.sync_copy(dst_hbm.at[s], raw_dst)

    lane = lax.iota(i32, 16)

    def build(m, _):
        r = m // (K // 16)
        col = (m % (K // 16)) * 16
        vs = raw_src[r, pl.ds(col, 16)]
        vd = raw_dst[r, pl.ds(col, 16)]
        valid = (m * 16 + lane) < EPT
        src2[r, pl.ds(col, 16)] = jnp.where(valid, vs, 0)
        dst2[r, pl.ds(col, 16)] = jnp.where(valid, vd, TRASH)
        return 0

    lax.fori_loop(0, EPAD // 16, build, 0)

    @pl.when(c == 0)
    def _publish():
        pltpu.sync_copy(src2, srcl_hbm.at[s])
        pltpu.sync_copy(dst2, dstl_hbm.at[s])


@functools.partial(
    pl.kernel,
    out_type=jax.ShapeDtypeStruct((ACCR, CW), f32),
    mesh=_mesh,
    scratch_types=[
        pltpu.VMEM((NBAT, K), i32),
        pltpu.VMEM((K, CW), f32),
        pltpu.VMEM_SHARED((ACCR, CW), f32),
    ],
)
def _deg(dstl_hbm, z_hbm, deg_hbm, dst2, ones_v, acc):
    c = lax.axis_index("c")
    s = lax.axis_index("s")

    pltpu.sync_copy(dstl_hbm.at[s], dst2)

    def orow(r, _):
        for j in range(CW // 16):
            ones_v[r, pl.ds(16 * j, 16)] = jnp.ones((16,), f32)
        return 0
    lax.fori_loop(0, K, orow, 0)

    pltpu.sync_copy(z_hbm.at[pl.ds(RPT * s, RPT)], acc.at[pl.ds(RPT * s, RPT)])
    plsc.subcore_barrier()

    def body(b, _):
        pltpu.sync_copy(ones_v, acc.at[dst2.at[b]], add=True)
        return 0
    lax.fori_loop(0, NBAT, body, 0)

    plsc.subcore_barrier()

    @pl.when(c == 0)
    def _out():
        pltpu.sync_copy(acc.at[pl.ds(RPT * s, RPT)], deg_hbm.at[pl.ds(RPT * s, RPT)])


@functools.partial(
    pl.kernel,
    out_type=jax.ShapeDtypeStruct((NC, ACCR, CW), f32),
    mesh=_mesh,
    scratch_types=[
        pltpu.VMEM((NBAT, K), i32),
        pltpu.VMEM((NBAT, K), i32),
        pltpu.VMEM((K, CW), f32),
        pltpu.VMEM_SHARED((ACCR, CW), f32),
        pltpu.SemaphoreType.DMA,
    ],
)
def _agg(g_hbm, srcl_hbm, dstl_hbm, z_hbm, out_hbm, src2, dst2, rows0, acc, sem0):
    c = lax.axis_index("c")
    s = lax.axis_index("s")

    pltpu.sync_copy(srcl_hbm.at[s], src2)
    pltpu.sync_copy(dstl_hbm.at[s], dst2)
    pltpu.sync_copy(z_hbm.at[pl.ds(RPT * s, RPT)], acc.at[pl.ds(RPT * s, RPT)])
    plsc.subcore_barrier()

    def body(b, _):
        @pl.when(c == 0)
        def _g0():
            pltpu.async_copy(g_hbm.at[0].at[src2.at[b]], rows0, sem0).wait()

        @pl.when(c == 1)
        def _g1():
            pltpu.async_copy(g_hbm.at[1].at[src2.at[b]], rows0, sem0).wait()

        pltpu.sync_copy(rows0, acc.at[dst2.at[b]], add=True)
        return 0
    lax.fori_loop(0, NBAT, body, 0)

    plsc.subcore_barrier()
    pltpu.sync_copy(acc.at[pl.ds(RPT * s, RPT)], out_hbm.at[c, pl.ds(RPT * s, RPT)])


def _enc_body(deg_ref, x_ref, WeT_ref, be_ref, Wc1T_ref, g1_ref, dinv_ref):
    dinv = lax.rsqrt(deg_ref[:, 0:1] + 1.0)
    h = jnp.dot(x_ref[...], WeT_ref[...], preferred_element_type=f32) + be_ref[...]
    h = jnp.maximum(h, 0.0)
    G = jnp.dot(h, Wc1T_ref[...], preferred_element_type=f32) * dinv
    g1_ref[0] = G[:, :CW]
    g1_ref[1] = G[:, CW:]
    dinv_ref[...] = dinv


_RB = 2000

_enc = pl.pallas_call(
    _enc_body,
    grid=(N // _RB,),
    in_specs=[
        pl.BlockSpec((_RB, CW), lambda i: (i, 0)),
        pl.BlockSpec((_RB, D), lambda i: (i, 0)),
        pl.BlockSpec((D, D), lambda i: (0, 0)),
        pl.BlockSpec((1, D), lambda i: (0, 0)),
        pl.BlockSpec((D, D), lambda i: (0, 0)),
    ],
    out_specs=[
        pl.BlockSpec((2, _RB, CW), lambda i: (0, i, 0)),
        pl.BlockSpec((_RB, 1), lambda i: (i, 0)),
    ],
    out_shape=[
        jax.ShapeDtypeStruct((2, N, CW), f32),
        jax.ShapeDtypeStruct((N, 1), f32),
    ],
)

_MB = 1000


def _mid_body(S0_ref, S1_ref, g_ref, dinv_ref, bc_ref, WuT_ref, bu_ref,
              WcnT_ref, out_ref):
    dinv = dinv_ref[...]
    A_lo = (S0_ref[0] + g_ref[0]) * dinv + bc_ref[:, :CW]
    A_hi = (S1_ref[0] + g_ref[1]) * dinv + bc_ref[:, CW:]
    U = (jnp.dot(A_lo, WuT_ref[:CW, :], preferred_element_type=f32)
         + jnp.dot(A_hi, WuT_ref[CW:, :], preferred_element_type=f32)
         + bu_ref[...])
    U = jnp.maximum(U, 0.0)
    G = jnp.dot(U, WcnT_ref[...], preferred_element_type=f32) * dinv
    out_ref[0] = G[:, :CW]
    out_ref[1] = G[:, CW:]


def _fin_body(S0_ref, S1_ref, g_ref, dinv_ref, bc_ref, WuT_ref, bu_ref, out_ref):
    dinv = dinv_ref[...]
    A_lo = (S0_ref[0] + g_ref[0]) * dinv + bc_ref[:, :CW]
    A_hi = (S1_ref[0] + g_ref[1]) * dinv + bc_ref[:, CW:]
    U = (jnp.dot(A_lo, WuT_ref[:CW, :], preferred_element_type=f32)
         + jnp.dot(A_hi, WuT_ref[CW:, :], preferred_element_type=f32)
         + bu_ref[...])
    out_ref[...] = jnp.maximum(U, 0.0)


def _mk_mid(final):
    body = _fin_body if final else _mid_body
    in_specs = [
        pl.BlockSpec((1, _MB, CW), lambda i: (0, i, 0)),
        pl.BlockSpec((1, _MB, CW), lambda i: (1, i, 0)),
        pl.BlockSpec((2, _MB, CW), lambda i: (0, i, 0)),
        pl.BlockSpec((_MB, 1), lambda i: (i, 0)),
        pl.BlockSpec((1, D), lambda i: (0, 0)),
        pl.BlockSpec((D, D), lambda i: (0, 0)),
        pl.BlockSpec((1, D), lambda i: (0, 0)),
    ]
    if not final:
        in_specs.append(pl.BlockSpec((D, D), lambda i: (0, 0)))
    return pl.pallas_call(
        body,
        grid=(N // _MB,),
        in_specs=in_specs,
        out_specs=(pl.BlockSpec((_MB, D), lambda i: (i, 0)) if final
                   else pl.BlockSpec((2, _MB, CW), lambda i: (0, i, 0))),
        out_shape=(jax.ShapeDtypeStruct((N, D), f32) if final
                   else jax.ShapeDtypeStruct((2, N, CW), f32)),
    )


_mid = _mk_mid(False)
_fin = _mk_mid(True)


def kernel(x, edge_index, We, be, Wc1, bc1, Wu1, bu1, Wc2, bc2, Wu2, bu2,
           Wc3, bc3, Wu3, bu3):
    ei = jnp.pad(edge_index.reshape(2, NS, EPT), ((0, 0), (0, 0), (0, EPAD - EPT)))
    src = ei[0].reshape(NS, NBAT, K)
    dst = ei[1].reshape(NS, NBAT, K)
    z = jnp.zeros((ACCR, CW), f32)

    srcl, dstl = _prep(src, dst)
    deg = _deg(dstl, z)

    g1, dinv = _enc(deg[:N], x, We.T, be[None], Wc1.T)

    S1 = _agg(g1, srcl, dstl, z)
    g2 = _mid(S1, S1, g1, dinv, bc1[None], Wu1.T, bu1[None], Wc2.T)

    S2 = _agg(g2, srcl, dstl, z)
    g3 = _mid(S2, S2, g2, dinv, bc2[None], Wu2.T, bu2[None], Wc3.T)

    S3 = _agg(g3, srcl, dstl, z)
    return _fin(S3, S3, g3, dinv, bc3[None], Wu3.T, bu3[None])
